# Initial kernel scaffold; baseline (speedup 1.0000x reference)
import jax
import jax.numpy as jnp
from jax.experimental import pallas as pl


def kernel(x, base_table, lora_A, lora_B):
    raise NotImplementedError("write your pallas kernel here")


# speedup vs baseline: 2.1655x; 2.1655x over previous
import jax
import jax.numpy as jnp
from jax import lax
from jax.experimental import pallas as pl
from jax.experimental.pallas import tpu as pltpu
from jax.experimental.pallas import tpu_sc as plsc

NUM_EMB = 1000000
D = 64
R = 8
SCALE = 16 / 8

NC = 2
NS = 16
NW = NC * NS
L = 16

B_TOTAL = 4096 * 50
B_W = B_TOTAL // NW
GROUP = 128
CHUNK = 640
G_PER_CHUNK = CHUNK // GROUP
N_CHUNKS = B_W // CHUNK


def _sc_body(x_hbm, a_hbm, base_hbm, lora_hbm, out_hbm,
             idx_v, rows_v, lrows_v, a_v, sem):
    wid = lax.axis_index("s") * NC + lax.axis_index("c")
    wbase = wid * B_W

    pltpu.sync_copy(x_hbm.at[pl.ds(wbase, B_W)], idx_v)
    pltpu.sync_copy(a_hbm, a_v)

    a_regs = [[a_v[r, pl.ds(dv * L, L)] for dv in range(4)]
              for r in range(R)]

    for k in range(N_CHUNKS):
        cbase = k * CHUNK
        copies = []
        for g in range(G_PER_CHUNK):
            isl = idx_v.at[pl.ds(cbase + g * GROUP, GROUP)]
            copies.append(pltpu.async_copy(
                base_hbm.at[isl], rows_v.at[pl.ds(g * GROUP, GROUP)], sem))
            copies.append(pltpu.async_copy(
                lora_hbm.at[isl], lrows_v.at[pl.ds(g * GROUP, GROUP)], sem))
        for c in copies:
            c.wait()

        def row_body(i, carry):
            bv = lrows_v[i, :]
            for dv in range(4):
                acc = rows_v[i, pl.ds(dv * L, L)]
                for r in range(R):
                    acc = acc + bv[r] * a_regs[r][dv]
                rows_v[i, pl.ds(dv * L, L)] = acc
            return carry

        lax.fori_loop(0, CHUNK, row_body, 0)

        pltpu.sync_copy(rows_v, out_hbm.at[pl.ds(wbase + cbase, CHUNK)])


def kernel(x, base_table, lora_A, lora_B):
    xf = x.reshape(-1)
    a_scaled = lora_A * SCALE
    lora_b16 = jnp.pad(lora_B, ((0, 0), (0, 2 * R - R)))

    mesh = plsc.VectorSubcoreMesh(core_axis_name="c", subcore_axis_name="s",
                                  num_cores=NC, num_subcores=NS)
    out = pl.kernel(
        _sc_body,
        out_type=jax.ShapeDtypeStruct((B_TOTAL, D), jnp.float32),
        mesh=mesh,
        compiler_params=pltpu.CompilerParams(use_tc_tiling_on_sc=False),
        scratch_types=[
            pltpu.VMEM((B_W,), jnp.int32),
            pltpu.VMEM((CHUNK, D), jnp.float32),
            pltpu.VMEM((CHUNK, 2 * R), jnp.float32),
            pltpu.VMEM((R, D), jnp.float32),
            pltpu.SemaphoreType.DMA,
        ],
    )(xf, a_scaled, base_table, lora_b16)
    return out.reshape(x.shape[0], x.shape[1], D)


# speedup vs baseline: 2.7624x; 1.2756x over previous
import jax
import jax.numpy as jnp
from jax import lax
from jax.experimental import pallas as pl
from jax.experimental.pallas import tpu as pltpu
from jax.experimental.pallas import tpu_sc as plsc

NUM_EMB = 1000000
D = 64
R = 8
SCALE = 16 / 8

NC = 2
NS = 16
NW = NC * NS
L = 16

B_TOTAL = 4096 * 50
B_W = B_TOTAL // NW
GROUP = 128
CHUNK = 640
G_PER_CHUNK = CHUNK // GROUP
N_CHUNKS = B_W // CHUNK


def _sc_body(x_hbm, a_hbm, base_hbm, lora_hbm, out_hbm,
             idx_v, rows_v, lrows_v, lflat_v, lsh_v, a_v, sem):
    sid = lax.axis_index("s")
    wid = sid * NC + lax.axis_index("c")
    wbase = wid * B_W

    pltpu.sync_copy(x_hbm.at[pl.ds(wbase, B_W)], idx_v)
    pltpu.sync_copy(a_hbm, a_v)

    a_regs = [[a_v[r, pl.ds(dv * L, L)] for dv in range(4)]
              for r in range(R)]

    for k in range(N_CHUNKS):
        cbase = k * CHUNK
        copies = []
        for g in range(G_PER_CHUNK):
            isl = idx_v.at[pl.ds(cbase + g * GROUP, GROUP)]
            copies.append(pltpu.async_copy(
                base_hbm.at[isl], rows_v.at[pl.ds(g * GROUP, GROUP)], sem))
            copies.append(pltpu.async_copy(
                lora_hbm.at[isl], lrows_v.at[pl.ds(g * GROUP, GROUP)], sem))
        for c in copies:
            c.wait()

        pltpu.sync_copy(lrows_v, lsh_v.at[sid])
        pltpu.sync_copy(lsh_v.at[sid],
                        lflat_v.at[pl.ds(0, CHUNK), pl.ds(0, R)])

        def row_body(i, carry):
            bv = lflat_v[i, :]
            for dv in range(4):
                acc = rows_v[i, pl.ds(dv * L, L)]
                for r in range(R):
                    acc = acc + bv[r] * a_regs[r][dv]
                rows_v[i, pl.ds(dv * L, L)] = acc
            return carry

        lax.fori_loop(0, CHUNK, row_body, 0)

        pltpu.sync_copy(rows_v, out_hbm.at[pl.ds(wbase + cbase, CHUNK)])


def kernel(x, base_table, lora_A, lora_B):
    xf = x.reshape(-1)
    a_scaled = lora_A * SCALE

    mesh = plsc.VectorSubcoreMesh(core_axis_name="c", subcore_axis_name="s",
                                  num_cores=NC, num_subcores=NS)
    out = pl.kernel(
        _sc_body,
        out_type=jax.ShapeDtypeStruct((B_TOTAL, D), jnp.float32),
        mesh=mesh,
        compiler_params=pltpu.CompilerParams(use_tc_tiling_on_sc=False),
        scratch_types=[
            pltpu.VMEM((B_W,), jnp.int32),
            pltpu.VMEM((CHUNK, D), jnp.float32),
            pltpu.VMEM((CHUNK, R), jnp.float32),
            pltpu.VMEM((CHUNK, 2 * R), jnp.float32),
            pltpu.VMEM_SHARED((NS, CHUNK, R), jnp.float32),
            pltpu.VMEM((R, D), jnp.float32),
            pltpu.SemaphoreType.DMA,
        ],
    )(xf, a_scaled, base_table, lora_B)
    return out.reshape(x.shape[0], x.shape[1], D)
